# baseline (speedup 1.0000x reference)
import functools

import jax
import jax.numpy as jnp
from jax import lax
from jax.experimental import pallas as pl
from jax.experimental.pallas import tpu as pltpu
from jax.experimental.pallas import tpu_sc as plsc

B = 200000
F_DIM = 128
E_DIM = 64
OUT_DIM = F_DIM + E_DIM

NW = 32
CHUNK = 320
NCHUNKS = B // CHUNK
CPW = -(-NCHUNKS // NW)
GSUB = 128


def _sc_fused(ids, feats, emb128):
    mesh = plsc.VectorSubcoreMesh(core_axis_name="c", subcore_axis_name="s")

    @functools.partial(
        pl.kernel,
        mesh=mesh,
        out_type=jax.ShapeDtypeStruct((B, OUT_DIM), jnp.float32),
        scratch_types=[
            pltpu.VMEM((CHUNK,), jnp.int32),
            pltpu.VMEM((CHUNK, F_DIM), jnp.float32),
            pltpu.SemaphoreType.DMA,
            pltpu.SemaphoreType.DMA,
        ],
    )
    def k(ids_hbm, feats_hbm, emb_hbm, out_hbm, idx_v, rows_v, sem_g, sem_o):
        wid = lax.axis_index("s") * 2 + lax.axis_index("c")
        col1 = lax.mul(jnp.int32(F_DIM), jnp.int32(1))

        def step(i, _):
            ci = wid + i * NW

            @pl.when(ci < NCHUNKS)
            def _():
                base = ci * CHUNK
                pltpu.sync_copy(ids_hbm.at[pl.ds(base, CHUNK)], idx_v)
                feat_cp = pltpu.async_copy(
                    feats_hbm.at[pl.ds(base, CHUNK), :],
                    out_hbm.at[pl.ds(base, CHUNK), pl.ds(0, F_DIM)], sem_o)
                gathers = []
                for s in range(0, CHUNK, GSUB):
                    n = min(GSUB, CHUNK - s)
                    gathers.append(pltpu.async_copy(
                        emb_hbm.at[idx_v.at[pl.ds(s, n)]],
                        rows_v.at[pl.ds(s, n)], sem_g))
                for g in gathers:
                    g.wait()
                w_emb = pltpu.async_copy(
                    rows_v,
                    out_hbm.at[pl.ds(base, CHUNK), pl.ds(col1, F_DIM)],
                    sem_g)
                w_emb.wait()
                feat_cp.wait()
            return ()

        lax.fori_loop(0, CPW, step, ())

    return k(ids, feats, emb128)


def kernel(ids, feats, hop_idx, emb_W):
    n_nodes = emb_W.shape[0] - 1
    gather_ids = jnp.where(hop_idx > 0, ids,
                           jnp.full_like(ids, n_nodes)).astype(jnp.int32)
    emb128 = jnp.pad(emb_W, ((0, 0), (0, F_DIM - E_DIM)))
    return _sc_fused(gather_ids, feats, emb128)


# speedup vs baseline: 7.0234x; 7.0234x over previous
import functools

import jax
import jax.numpy as jnp
from jax import lax
from jax.experimental import pallas as pl
from jax.experimental.pallas import tpu as pltpu
from jax.experimental.pallas import tpu_sc as plsc

B = 200000
F_DIM = 128
E_DIM = 64
OUT_DIM = F_DIM + E_DIM

NW = 32
CHUNK = 320
NCHUNKS = B // CHUNK
CPW = -(-NCHUNKS // NW)
GSUB = 128

FE_ROWS = 1000


def _sc_gather_into_out(ids, emb128):
    mesh = plsc.VectorSubcoreMesh(core_axis_name="c", subcore_axis_name="s")

    @functools.partial(
        pl.kernel,
        mesh=mesh,
        out_type=jax.ShapeDtypeStruct((B, OUT_DIM), jnp.float32),
        scratch_types=[
            pltpu.VMEM((CHUNK,), jnp.int32),
            pltpu.VMEM((CHUNK, F_DIM), jnp.float32),
            pltpu.SemaphoreType.DMA,
        ],
    )
    def k(ids_hbm, emb_hbm, out_hbm, idx_v, rows_v, sem_g):
        wid = lax.axis_index("s") * 2 + lax.axis_index("c")
        col1 = lax.mul(jnp.int32(F_DIM), jnp.int32(1))

        def step(i, _):
            ci = wid + i * NW

            @pl.when(ci < NCHUNKS)
            def _():
                base = ci * CHUNK
                pltpu.sync_copy(ids_hbm.at[pl.ds(base, CHUNK)], idx_v)
                gathers = []
                for s in range(0, CHUNK, GSUB):
                    n = min(GSUB, CHUNK - s)
                    gathers.append(pltpu.async_copy(
                        emb_hbm.at[idx_v.at[pl.ds(s, n)]],
                        rows_v.at[pl.ds(s, n)], sem_g))
                for g in gathers:
                    g.wait()
                w_emb = pltpu.async_copy(
                    rows_v,
                    out_hbm.at[pl.ds(base, CHUNK), pl.ds(col1, F_DIM)],
                    sem_g)
                w_emb.wait()
            return ()

        lax.fori_loop(0, CPW, step, ())

    return k(ids, emb128)


def _tc_write_feats(out_sc, feats):
    def body(_, feats_ref, out_ref):
        out_ref[...] = feats_ref[...]

    return pl.pallas_call(
        body,
        grid=(B // FE_ROWS,),
        in_specs=[
            pl.BlockSpec(memory_space=pl.ANY),
            pl.BlockSpec((FE_ROWS, F_DIM), lambda i: (i, 0)),
        ],
        out_specs=pl.BlockSpec((FE_ROWS, F_DIM), lambda i: (i, 0)),
        out_shape=jax.ShapeDtypeStruct((B, OUT_DIM), jnp.float32),
        input_output_aliases={0: 0},
    )(out_sc, feats)


def kernel(ids, feats, hop_idx, emb_W):
    n_nodes = emb_W.shape[0] - 1
    gather_ids = jnp.where(hop_idx > 0, ids,
                           jnp.full_like(ids, n_nodes)).astype(jnp.int32)
    emb128 = jnp.pad(emb_W, ((0, 0), (0, F_DIM - E_DIM)))
    out_sc = _sc_gather_into_out(gather_ids, emb128)
    return _tc_write_feats(out_sc, feats)
